# baseline (speedup 1.0000x reference)
import functools

import jax
import jax.numpy as jnp
from jax import lax
from jax.experimental import pallas as pl
from jax.experimental.pallas import tpu as pltpu


def _se_kernel(x_ref, w1t_ref, w2_ref, o_ref, *, inv_hw):
    x = x_ref[...]
    bt, hw, c = x.shape

    pooled = jnp.sum(x, axis=1) * inv_hw

    h = jnp.maximum(
        lax.dot_general(pooled, w1t_ref[...],
                        dimension_numbers=(((1,), (1,)), ((), ())),
                        preferred_element_type=jnp.float32), 0.0)
    s = jax.nn.sigmoid(
        jnp.dot(h, w2_ref[...], preferred_element_type=jnp.float32))

    o_ref[...] = x * s.astype(x.dtype)[:, None, :]


def kernel(x, w1, w2):
    B, C, H, W = x.shape
    HW = H * W
    Cr = w1.shape[1]
    itemsize = x.dtype.itemsize

    x_cm = x.transpose(0, 2, 3, 1).reshape(B, HW, C)

    row_bytes = HW * C * itemsize
    Bt = 1
    for bt in (2, 4):
        if B % bt == 0 and bt * row_bytes <= (8 << 20):
            Bt = bt
    vmem_limit = int(min(4 * Bt * row_bytes + (8 << 20), 56 << 20))

    out_cm = pl.pallas_call(
        functools.partial(_se_kernel, inv_hw=1.0 / HW),
        out_shape=jax.ShapeDtypeStruct((B, HW, C), x.dtype),
        grid=(B // Bt,),
        in_specs=[
            pl.BlockSpec((Bt, HW, C), lambda g: (g, 0, 0)),
            pl.BlockSpec((Cr, C), lambda g: (0, 0)),
            pl.BlockSpec((Cr, C), lambda g: (0, 0)),
        ],
        out_specs=pl.BlockSpec((Bt, HW, C), lambda g: (g, 0, 0)),
        compiler_params=pltpu.CompilerParams(
            dimension_semantics=("parallel",),
            vmem_limit_bytes=vmem_limit),
        cost_estimate=pl.CostEstimate(
            flops=3 * B * C * HW + 4 * B * C * Cr,
            transcendentals=B * C,
            bytes_accessed=2 * B * C * HW * itemsize + (w1.size + w2.size) * 4),
    )(x_cm, w1.astype(jnp.float32).T, w2.astype(jnp.float32))

    return out_cm.reshape(B, H, W, C).transpose(0, 3, 1, 2)


# speedup vs baseline: 1.0563x; 1.0563x over previous
import functools

import jax
import jax.numpy as jnp
from jax import lax
from jax.experimental import pallas as pl
from jax.experimental.pallas import tpu as pltpu


def _se_kernel(x_ref, w1t_ref, w2_ref, o_ref, *, inv_hw):
    x = x_ref[...]
    bt, hw, c = x.shape

    pooled = jnp.sum(x, axis=1) * inv_hw

    h = jnp.maximum(
        lax.dot_general(pooled, w1t_ref[...],
                        dimension_numbers=(((1,), (1,)), ((), ())),
                        preferred_element_type=jnp.float32), 0.0)
    s = jax.nn.sigmoid(
        jnp.dot(h, w2_ref[...], preferred_element_type=jnp.float32))

    o_ref[...] = x * s.astype(x.dtype)[:, None, :]


def kernel(x, w1, w2):
    B, C, H, W = x.shape
    HW = H * W
    Cr = w1.shape[1]
    itemsize = x.dtype.itemsize

    x_cm = x.transpose(0, 2, 3, 1).reshape(B, HW, C)

    row_bytes = HW * C * itemsize
    Bt = 1
    for bt in (2, 4):
        if B % bt == 0 and bt * row_bytes <= (14 << 20):
            Bt = bt
    vmem_limit = int(min(4 * Bt * row_bytes + (8 << 20), 56 << 20))

    out_cm = pl.pallas_call(
        functools.partial(_se_kernel, inv_hw=1.0 / HW),
        out_shape=jax.ShapeDtypeStruct((B, HW, C), x.dtype),
        grid=(B // Bt,),
        in_specs=[
            pl.BlockSpec((Bt, HW, C), lambda g: (g, 0, 0)),
            pl.BlockSpec((Cr, C), lambda g: (0, 0)),
            pl.BlockSpec((Cr, C), lambda g: (0, 0)),
        ],
        out_specs=pl.BlockSpec((Bt, HW, C), lambda g: (g, 0, 0)),
        compiler_params=pltpu.CompilerParams(
            dimension_semantics=("parallel",),
            vmem_limit_bytes=vmem_limit),
        cost_estimate=pl.CostEstimate(
            flops=3 * B * C * HW + 4 * B * C * Cr,
            transcendentals=B * C,
            bytes_accessed=2 * B * C * HW * itemsize + (w1.size + w2.size) * 4),
    )(x_cm, w1.astype(jnp.float32).T, w2.astype(jnp.float32))

    return out_cm.reshape(B, H, W, C).transpose(0, 3, 1, 2)
